# baseline (speedup 1.0000x reference)
import functools

import jax
import jax.numpy as jnp
from jax import lax
from jax.experimental import pallas as pl
from jax.experimental.pallas import tpu as pltpu
from jax.experimental.pallas import tpu_sc as plsc

N = 10000
H = 128
NP = 10112
BLK = NP // 16
GRID = 16
NCORE = 2
NSUB = 16
NTILE = NCORE * NSUB
HALF = NP // 2
ACCR = 5120
TRASH = HALF
ZROW = NP - 8
CHUNK = 32
WBLK = 512


def _lrelu(v):
    return jnp.where(v > 0, v, 0.01 * v)


def _dot(a, b):
    return jnp.dot(a, b, preferred_element_type=jnp.float32,
                   precision=lax.Precision.HIGHEST)


def _row_mask(blk_rows):
    i = pl.program_id(0)
    rows = i * blk_rows + lax.broadcasted_iota(jnp.int32, (blk_rows, 1), 0)
    return (rows < N).astype(jnp.float32)


def _enc_body(x_ref, w0, b0, w1, b1, w2, b2, t_ref):
    h = _lrelu(_dot(x_ref[...], w0[...]) + b0[...])
    h = _lrelu(_dot(h, w1[...]) + b1[...])
    h = _dot(h, w2[...]) + b2[...]
    t_ref[...] = h * _row_mask(BLK)


def _wspec():
    return pl.BlockSpec((H, H), lambda i: (0, 0))


def _bspec():
    return pl.BlockSpec((1, H), lambda i: (0, 0))


def _rspec(width=H):
    return pl.BlockSpec((BLK, width), lambda i: (i, 0))


_enc = pl.pallas_call(
    _enc_body,
    grid=(GRID,),
    in_specs=[_rspec()] + [_wspec(), _bspec()] * 3,
    out_specs=_rspec(),
    out_shape=jax.ShapeDtypeStruct((NP, H), jnp.float32),
)


def _step_body(t_ref, g0_ref, g1_ref, deg_ref, wa, wb, eb,
               p0w, p0b, p1w, p1b, p2w, p2b, to_ref):
    xb = t_ref[...]
    g = g0_ref[...] + g1_ref[...]
    deg = deg_ref[...]
    m = _dot(deg * xb, wa[...]) + _dot(g + xb, wb[...]) + deg * eb[...]
    h = _lrelu(_dot(m, p0w[...]) + p0b[...])
    h = _lrelu(_dot(h, p1w[...]) + p1b[...])
    h = _dot(h, p2w[...]) + p2b[...]
    to_ref[...] = h * _row_mask(BLK)


_step = pl.pallas_call(
    _step_body,
    grid=(GRID,),
    in_specs=[_rspec(), _rspec(), _rspec(), _rspec(1),
              _wspec(), _wspec(), _bspec(),
              _wspec(), _bspec(), _wspec(), _bspec(), _wspec(), _bspec()],
    out_specs=_rspec(),
    out_shape=jax.ShapeDtypeStruct((NP, H), jnp.float32),
)


def _dec_body(t_ref, d0w, d0b, d1w, d1b, d2w, d2b, o_ref):
    h = _lrelu(_dot(t_ref[...], d0w[...]) + d0b[...])
    h = _lrelu(_dot(h, d1w[...]) + d1b[...])
    o_ref[...] = _dot(h, d2w[...]) + d2b[...]


def _make_dec(out_dim):
    return pl.pallas_call(
        _dec_body,
        grid=(GRID,),
        in_specs=[_rspec(), _wspec(), _bspec(), _wspec(), _bspec(),
                  pl.BlockSpec((H, out_dim), lambda i: (0, 0)),
                  pl.BlockSpec((1, out_dim), lambda i: (0, 0))],
        out_specs=_rspec(out_dim),
        out_shape=jax.ShapeDtypeStruct((NP, out_dim), jnp.float32),
    )


def _mesh():
    return plsc.VectorSubcoreMesh(core_axis_name="c", subcore_axis_name="s")


THR = 5064
GROW = 4


@functools.lru_cache(maxsize=None)
def _make_segsum(n_groups):
    gpt = n_groups // NSUB
    trows = HALF // 8
    arows = NP // NSUB

    @functools.partial(
        pl.kernel,
        out_type=jax.ShapeDtypeStruct((NCORE, NP, H), jnp.float32),
        mesh=_mesh(),
        scratch_types=[
            pltpu.VMEM((2 * GROW, 128), jnp.int32),
            pltpu.VMEM((CHUNK, H), jnp.float32),
            pltpu.VMEM((CHUNK, H), jnp.float32),
            pltpu.VMEM_SHARED((THR, H), jnp.float32),
            pltpu.VMEM_SHARED((NP, H), jnp.float32),
            pltpu.SemaphoreType.DMA,
            pltpu.SemaphoreType.DMA,
        ],
    )
    def segsum(table, ed, out, stg, rb0, rb1, tbl, acc, gs0, gs1):
        c = lax.axis_index("c")
        s = lax.axis_index("s")

        zero = jnp.zeros((16,), jnp.float32)

        def zrow(r, carry):
            for k in range(H // 16):
                rb0[r, pl.ds(k * 16, 16)] = zero
            return carry

        lax.fori_loop(0, CHUNK, zrow, 0)
        a0 = s * arows
        for t in range(arows // CHUNK):
            pltpu.sync_copy(rb0, acc.at[pl.ds(a0 + t * CHUNK, CHUNK)])
        pltpu.sync_copy(rb0.at[pl.ds(0, arows % CHUNK)],
                        acc.at[pl.ds(a0 + arows - arows % CHUNK,
                                     arows % CHUNK)])

        @pl.when(s < 8)
        def _():
            pltpu.sync_copy(table.at[pl.ds(c * HALF + s * trows, trows)],
                            tbl.at[pl.ds(s * trows, trows)])

        @pl.when(s == 8)
        def _():
            pltpu.sync_copy(table.at[pl.ds(NP - 8, 8)],
                            tbl.at[pl.ds(HALF, 8)])

        plsc.subcore_barrier()

        rbb = (rb0, rb1)
        gsb = (gs0, gs1)

        def start_g(r, kk, b):
            pltpu.make_async_copy(
                tbl.at[stg.at[r, pl.ds(kk * CHUNK, CHUNK)]],
                rbb[b], gsb[b]).start()

        def wait_g(r, kk, b):
            pltpu.make_async_copy(
                tbl.at[stg.at[r, pl.ds(kk * CHUNK, CHUNK)]],
                rbb[b], gsb[b]).wait()

        spg = GROW * (128 // CHUNK)

        def group(g, carry):
            pltpu.sync_copy(ed.at[c, s * gpt + g], stg)
            start_g(0, 0, 0)
            for t in range(spg):
                r, kk = t // (128 // CHUNK), t % (128 // CHUNK)
                b = t & 1
                wait_g(r, kk, b)
                if t + 1 < spg:
                    r2, kk2 = (t + 1) // (128 // CHUNK), (t + 1) % (128 // CHUNK)
                    start_g(r2, kk2, 1 - b)
                for h in range(CHUNK // 16):
                    dv = stg[GROW + r, pl.ds(kk * CHUNK + h * 16, 16)]
                    pltpu.sync_copy(rbb[b].at[pl.ds(h * 16, 16)],
                                    acc.at[dv], add=True)
            return carry

        lax.fori_loop(0, gpt, group, 0)
        plsc.subcore_barrier()

        pltpu.sync_copy(acc.at[pl.ds(a0, arows)],
                        out.at[c].at[pl.ds(a0, arows)])

    return segsum


NPD = 10240


@functools.lru_cache(maxsize=None)
def _make_deg(n_rows):
    rt = n_rows // NTILE
    rows_per_tile = NPD // NSUB

    @functools.partial(
        pl.kernel,
        out_type=jax.ShapeDtypeStruct((NCORE, NPD), jnp.int32),
        mesh=_mesh(),
        scratch_types=[
            pltpu.VMEM((rt, 128), jnp.int32),
            pltpu.VMEM((128,), jnp.int32),
            pltpu.VMEM((rows_per_tile,), jnp.int32),
            pltpu.VMEM_SHARED((NPD,), jnp.int32),
        ],
    )
    def degk(dst2, out, idx_d, ones_b, zb, accd):
        c = lax.axis_index("c")
        s = lax.axis_index("s")
        wid = c * NSUB + s

        zero = jnp.zeros((16,), jnp.int32)
        one = jnp.ones((16,), jnp.int32)

        def zrow(r, carry):
            zb[pl.ds(r * 16, 16)] = zero
            return carry

        lax.fori_loop(0, rows_per_tile // 16, zrow, 0)

        def orow(r, carry):
            ones_b[pl.ds(r * 16, 16)] = one
            return carry

        lax.fori_loop(0, 128 // 16, orow, 0)
        r0 = s * rows_per_tile
        pltpu.sync_copy(zb, accd.at[pl.ds(r0, rows_per_tile)])
        plsc.subcore_barrier()

        base = wid * rt
        pltpu.sync_copy(dst2.at[pl.ds(base, rt)], idx_d)

        def hbody(j, carry):
            pltpu.sync_copy(ones_b, accd.at[idx_d.at[j]], add=True)
            return carry

        lax.fori_loop(0, rt, hbody, 0)
        plsc.subcore_barrier()

        pltpu.sync_copy(accd.at[pl.ds(r0, rows_per_tile)], zb)
        pltpu.sync_copy(zb, out.at[c].at[pl.ds(r0, rows_per_tile)])

    return degk


def kernel(x, edge_index,
           enc_W0, enc_b0, enc_W1, enc_b1, enc_W2, enc_b2,
           edge_W, edge_b,
           proc_W0, proc_b0, proc_W1, proc_b1, proc_W2, proc_b2,
           dec_W0, dec_b0, dec_W1, dec_b1, dec_W2, dec_b2):
    e = edge_index.shape[1]
    align = 128 * NTILE * 8
    ep = ((e + align - 1) // align) * align
    n_rows = ep // 128

    src = edge_index[0].astype(jnp.int32)
    dst = edge_index[1].astype(jnp.int32)
    pad = jnp.full((ep - e,), N, dtype=jnp.int32)
    src2 = jnp.concatenate([src, pad]).reshape(n_rows, 128)
    dst2 = jnp.concatenate([dst, pad]).reshape(n_rows, 128)
    xp = jnp.pad(x, ((0, NP - N), (0, 0)))

    def b2d(b):
        return b.reshape(1, -1)

    srcl = []
    for ci in range(NCORE):
        inh = (src2 >= ci * HALF) & (src2 < (ci + 1) * HALF)
        srcl.append(jnp.where(inh, src2 - ci * HALF, HALF))
    n_groups = n_rows // GROW
    a = jnp.stack(srcl).reshape(NCORE, n_groups, GROW, 128)
    b = jnp.broadcast_to(dst2.reshape(1, n_groups, GROW, 128),
                         (NCORE, n_groups, GROW, 128))
    ed = jnp.concatenate([a, b], axis=2)

    dp = _make_deg(n_rows)(dst2)
    deg = (dp[0] + dp[1])[:NP].reshape(NP, 1).astype(jnp.float32) + 1.0

    t = _enc(xp, enc_W0, b2d(enc_b0), enc_W1, b2d(enc_b1),
             enc_W2, b2d(enc_b2))

    wa = edge_W[:H]
    wb = edge_W[H:2 * H]
    segsum = _make_segsum(n_groups)
    for _ in range(3):
        g = segsum(t, ed)
        t = _step(t, g[0], g[1], deg, wa, wb, b2d(edge_b),
                  proc_W0, b2d(proc_b0), proc_W1, b2d(proc_b1),
                  proc_W2, b2d(proc_b2))

    out = _make_dec(dec_W2.shape[1])(t, dec_W0, b2d(dec_b0),
                                     dec_W1, b2d(dec_b1),
                                     dec_W2, b2d(dec_b2))
    return out[:N]


# speedup vs baseline: 1.0259x; 1.0259x over previous
import functools

import jax
import jax.numpy as jnp
from jax import lax
from jax.experimental import pallas as pl
from jax.experimental.pallas import tpu as pltpu
from jax.experimental.pallas import tpu_sc as plsc

N = 10000
H = 128
NP = 10112
BLK = NP // 16
GRID = 16
NCORE = 2
NSUB = 16
NTILE = NCORE * NSUB
HALF = NP // 2
ACCR = 5120
TRASH = HALF
ZROW = NP - 8
CHUNK = 32
WBLK = 512


def _lrelu(v):
    return jnp.where(v > 0, v, 0.01 * v)


def _dot(a, b):
    return jnp.dot(a, b, preferred_element_type=jnp.float32,
                   precision=lax.Precision.HIGHEST)


def _row_mask(blk_rows):
    i = pl.program_id(0)
    rows = i * blk_rows + lax.broadcasted_iota(jnp.int32, (blk_rows, 1), 0)
    return (rows < N).astype(jnp.float32)


def _enc_body(x_ref, w0, b0, w1, b1, w2, b2, t_ref):
    h = _lrelu(_dot(x_ref[...], w0[...]) + b0[...])
    h = _lrelu(_dot(h, w1[...]) + b1[...])
    h = _dot(h, w2[...]) + b2[...]
    t_ref[...] = h * _row_mask(BLK)


def _wspec():
    return pl.BlockSpec((H, H), lambda i: (0, 0))


def _bspec():
    return pl.BlockSpec((1, H), lambda i: (0, 0))


def _rspec(width=H):
    return pl.BlockSpec((BLK, width), lambda i: (i, 0))


_enc = pl.pallas_call(
    _enc_body,
    grid=(GRID,),
    in_specs=[_rspec()] + [_wspec(), _bspec()] * 3,
    out_specs=_rspec(),
    out_shape=jax.ShapeDtypeStruct((NP, H), jnp.float32),
)


def _step_body(t_ref, g0_ref, g1_ref, deg_ref, wa, wb, eb,
               p0w, p0b, p1w, p1b, p2w, p2b, to_ref):
    xb = t_ref[...]
    g = g0_ref[...] + g1_ref[...]
    deg = deg_ref[...]
    m = _dot(deg * xb, wa[...]) + _dot(g + xb, wb[...]) + deg * eb[...]
    h = _lrelu(_dot(m, p0w[...]) + p0b[...])
    h = _lrelu(_dot(h, p1w[...]) + p1b[...])
    h = _dot(h, p2w[...]) + p2b[...]
    to_ref[...] = h * _row_mask(BLK)


_step = pl.pallas_call(
    _step_body,
    grid=(GRID,),
    in_specs=[_rspec(), _rspec(), _rspec(), _rspec(1),
              _wspec(), _wspec(), _bspec(),
              _wspec(), _bspec(), _wspec(), _bspec(), _wspec(), _bspec()],
    out_specs=_rspec(),
    out_shape=jax.ShapeDtypeStruct((NP, H), jnp.float32),
)


def _dec_body(t_ref, d0w, d0b, d1w, d1b, d2w, d2b, o_ref):
    h = _lrelu(_dot(t_ref[...], d0w[...]) + d0b[...])
    h = _lrelu(_dot(h, d1w[...]) + d1b[...])
    o_ref[...] = _dot(h, d2w[...]) + d2b[...]


def _make_dec(out_dim):
    return pl.pallas_call(
        _dec_body,
        grid=(GRID,),
        in_specs=[_rspec(), _wspec(), _bspec(), _wspec(), _bspec(),
                  pl.BlockSpec((H, out_dim), lambda i: (0, 0)),
                  pl.BlockSpec((1, out_dim), lambda i: (0, 0))],
        out_specs=_rspec(out_dim),
        out_shape=jax.ShapeDtypeStruct((NP, out_dim), jnp.float32),
    )


def _mesh():
    return plsc.VectorSubcoreMesh(core_axis_name="c", subcore_axis_name="s")


THR = 5064
GROW = 4


@functools.lru_cache(maxsize=None)
def _make_segsum(n_groups):
    gpt = n_groups // NSUB
    trows = HALF // 8
    arows = NP // NSUB

    @functools.partial(
        pl.kernel,
        out_type=jax.ShapeDtypeStruct((NCORE, NP, H), jnp.float32),
        mesh=_mesh(),
        scratch_types=[
            pltpu.VMEM((2 * GROW, 128), jnp.int32),
            pltpu.VMEM((CHUNK, H), jnp.float32),
            pltpu.VMEM((CHUNK, H), jnp.float32),
            pltpu.VMEM_SHARED((THR, H), jnp.float32),
            pltpu.VMEM_SHARED((NP, H), jnp.float32),
            pltpu.SemaphoreType.DMA,
            pltpu.SemaphoreType.DMA,
            pltpu.SemaphoreType.DMA,
            pltpu.SemaphoreType.DMA,
        ],
    )
    def segsum(table, ed, out, stg, rb0, rb1, tbl, acc, gs0, gs1, sc0, sc1):
        c = lax.axis_index("c")
        s = lax.axis_index("s")

        zero = jnp.zeros((16,), jnp.float32)

        def zrow(r, carry):
            for k in range(H // 16):
                rb0[r, pl.ds(k * 16, 16)] = zero
            return carry

        lax.fori_loop(0, CHUNK, zrow, 0)
        a0 = s * arows
        for t in range(arows // CHUNK):
            pltpu.sync_copy(rb0, acc.at[pl.ds(a0 + t * CHUNK, CHUNK)])
        pltpu.sync_copy(rb0.at[pl.ds(0, arows % CHUNK)],
                        acc.at[pl.ds(a0 + arows - arows % CHUNK,
                                     arows % CHUNK)])

        @pl.when(s < 8)
        def _():
            pltpu.sync_copy(table.at[pl.ds(c * HALF + s * trows, trows)],
                            tbl.at[pl.ds(s * trows, trows)])

        @pl.when(s == 8)
        def _():
            pltpu.sync_copy(table.at[pl.ds(NP - 8, 8)],
                            tbl.at[pl.ds(HALF, 8)])

        plsc.subcore_barrier()

        rbb = (rb0, rb1)
        gsb = (gs0, gs1)
        scb = (sc0, sc1)

        def start_g(r, kk, b):
            pltpu.make_async_copy(
                tbl.at[stg.at[r, pl.ds(kk * CHUNK, CHUNK)]],
                rbb[b], gsb[b]).start()

        def wait_g(r, kk, b):
            pltpu.make_async_copy(
                tbl.at[stg.at[r, pl.ds(kk * CHUNK, CHUNK)]],
                rbb[b], gsb[b]).wait()

        spg = GROW * (128 // CHUNK)

        def group(g, carry):
            pltpu.sync_copy(ed.at[c, s * gpt + g], stg)
            start_g(0, 0, 0)
            pend = {0: [], 1: []}
            for t in range(spg):
                r, kk = t // (128 // CHUNK), t % (128 // CHUNK)
                b = t & 1
                wait_g(r, kk, b)
                if t + 1 < spg:
                    r2 = (t + 1) // (128 // CHUNK)
                    kk2 = (t + 1) % (128 // CHUNK)
                    for d in pend[1 - b]:
                        d.wait()
                    pend[1 - b] = []
                    start_g(r2, kk2, 1 - b)
                for h in range(CHUNK // 16):
                    dv = stg[GROW + r, pl.ds(kk * CHUNK + h * 16, 16)]
                    pend[b].append(pltpu.async_copy(
                        rbb[b].at[pl.ds(h * 16, 16)], acc.at[dv], scb[b],
                        add=True))
            for b in (0, 1):
                for d in pend[b]:
                    d.wait()
            return carry

        lax.fori_loop(0, gpt, group, 0)
        plsc.subcore_barrier()

        pltpu.sync_copy(acc.at[pl.ds(a0, arows)],
                        out.at[c].at[pl.ds(a0, arows)])

    return segsum


NPD = 10240


@functools.lru_cache(maxsize=None)
def _make_deg(n_rows):
    rt = n_rows // NTILE
    rows_per_tile = NPD // NSUB

    @functools.partial(
        pl.kernel,
        out_type=jax.ShapeDtypeStruct((NCORE, NPD), jnp.int32),
        mesh=_mesh(),
        scratch_types=[
            pltpu.VMEM((rt, 128), jnp.int32),
            pltpu.VMEM((128,), jnp.int32),
            pltpu.VMEM((rows_per_tile,), jnp.int32),
            pltpu.VMEM_SHARED((NPD,), jnp.int32),
        ],
    )
    def degk(dst2, out, idx_d, ones_b, zb, accd):
        c = lax.axis_index("c")
        s = lax.axis_index("s")
        wid = c * NSUB + s

        zero = jnp.zeros((16,), jnp.int32)
        one = jnp.ones((16,), jnp.int32)

        def zrow(r, carry):
            zb[pl.ds(r * 16, 16)] = zero
            return carry

        lax.fori_loop(0, rows_per_tile // 16, zrow, 0)

        def orow(r, carry):
            ones_b[pl.ds(r * 16, 16)] = one
            return carry

        lax.fori_loop(0, 128 // 16, orow, 0)
        r0 = s * rows_per_tile
        pltpu.sync_copy(zb, accd.at[pl.ds(r0, rows_per_tile)])
        plsc.subcore_barrier()

        base = wid * rt
        pltpu.sync_copy(dst2.at[pl.ds(base, rt)], idx_d)

        def hbody(j, carry):
            pltpu.sync_copy(ones_b, accd.at[idx_d.at[j]], add=True)
            return carry

        lax.fori_loop(0, rt, hbody, 0)
        plsc.subcore_barrier()

        pltpu.sync_copy(accd.at[pl.ds(r0, rows_per_tile)], zb)
        pltpu.sync_copy(zb, out.at[c].at[pl.ds(r0, rows_per_tile)])

    return degk


def kernel(x, edge_index,
           enc_W0, enc_b0, enc_W1, enc_b1, enc_W2, enc_b2,
           edge_W, edge_b,
           proc_W0, proc_b0, proc_W1, proc_b1, proc_W2, proc_b2,
           dec_W0, dec_b0, dec_W1, dec_b1, dec_W2, dec_b2):
    e = edge_index.shape[1]
    align = 128 * NTILE * 8
    ep = ((e + align - 1) // align) * align
    n_rows = ep // 128

    src = edge_index[0].astype(jnp.int32)
    dst = edge_index[1].astype(jnp.int32)
    pad = jnp.full((ep - e,), N, dtype=jnp.int32)
    src2 = jnp.concatenate([src, pad]).reshape(n_rows, 128)
    dst2 = jnp.concatenate([dst, pad]).reshape(n_rows, 128)
    xp = jnp.pad(x, ((0, NP - N), (0, 0)))

    def b2d(b):
        return b.reshape(1, -1)

    srcl = []
    for ci in range(NCORE):
        inh = (src2 >= ci * HALF) & (src2 < (ci + 1) * HALF)
        srcl.append(jnp.where(inh, src2 - ci * HALF, HALF))
    n_groups = n_rows // GROW
    a = jnp.stack(srcl).reshape(NCORE, n_groups, GROW, 128)
    b = jnp.broadcast_to(dst2.reshape(1, n_groups, GROW, 128),
                         (NCORE, n_groups, GROW, 128))
    ed = jnp.concatenate([a, b], axis=2)

    dp = _make_deg(n_rows)(dst2)
    deg = (dp[0] + dp[1])[:NP].reshape(NP, 1).astype(jnp.float32) + 1.0

    t = _enc(xp, enc_W0, b2d(enc_b0), enc_W1, b2d(enc_b1),
             enc_W2, b2d(enc_b2))

    wa = edge_W[:H]
    wb = edge_W[H:2 * H]
    segsum = _make_segsum(n_groups)
    for _ in range(3):
        g = segsum(t, ed)
        t = _step(t, g[0], g[1], deg, wa, wb, b2d(edge_b),
                  proc_W0, b2d(proc_b0), proc_W1, b2d(proc_b1),
                  proc_W2, b2d(proc_b2))

    out = _make_dec(dec_W2.shape[1])(t, dec_W0, b2d(dec_b0),
                                     dec_W1, b2d(dec_b1),
                                     dec_W2, b2d(dec_b2))
    return out[:N]
